# baseline (speedup 1.0000x reference)
import jax
import jax.numpy as jnp
from jax import lax
from jax.experimental import pallas as pl
from jax.experimental.pallas import tpu as pltpu
from jax.experimental.pallas import tpu_sc as plsc

DEG = 64
HIDDEN = 128
LANES = 16
N_CHUNKS = HIDDEN // LANES
NW = 4
ROWS_W = DEG // NW


def _sc_body(v_hbm, sel_hbm, out_hbm, idx_v, rows_v, out_v, sem):
    s = lax.axis_index("s")
    w = s

    @pl.when(s < NW)
    def _():
        pltpu.sync_copy(sel_hbm.at[pl.ds(w * ROWS_W, ROWS_W)], idx_v)
        pltpu.async_copy(v_hbm.at[idx_v], rows_v, sem).wait()
        def add_row(i, accs):
            return tuple(accs[j] + rows_v[i, pl.ds(j * LANES, LANES)]
                         for j in range(N_CHUNKS))

        accs = lax.fori_loop(
            0, ROWS_W, add_row,
            tuple(jnp.zeros((LANES,), jnp.float32) for _ in range(N_CHUNKS)))
        for j in range(N_CHUNKS):
            out_v[pl.ds(j * LANES, LANES)] = accs[j]
        pltpu.sync_copy(out_v, out_hbm.at[w])


def _sc_gather_sum(v, sel_masked):
    mesh = plsc.VectorSubcoreMesh(core_axis_name="c", subcore_axis_name="s",
                                  num_cores=1, num_subcores=NW)
    return pl.kernel(
        _sc_body,
        out_type=jax.ShapeDtypeStruct((NW, HIDDEN), jnp.float32),
        mesh=mesh,
        scratch_types=[
            pltpu.VMEM((ROWS_W,), jnp.int32),
            pltpu.VMEM((ROWS_W, HIDDEN), jnp.float32),
            pltpu.VMEM((HIDDEN,), jnp.float32),
            pltpu.SemaphoreType.DMA,
        ],
    )(v, sel_masked)


def _tc_body(parts_ref, v0_ref, times_ref, thr_ref, w0_ref, b0_ref, W_ref,
             B_ref, Wv_ref, bv_ref, term_ref):
    parts = parts_ref[...]
    row_sum = jnp.sum(parts, axis=0, keepdims=True)
    v0 = v0_ref[0:1, :]
    times = times_ref[...]
    maskf = (times <= thr_ref[...]).astype(jnp.float32)
    lin = times * w0_ref[...] + b0_ref[...]
    per = jnp.sin(times * W_ref[...] + B_ref[...])
    z = jnp.concatenate([lin, per], axis=1)
    zm = z * maskf
    t_v_sum = jnp.sum(
        jnp.dot(zm, Wv_ref[...], preferred_element_type=jnp.float32),
        axis=0, keepdims=True)
    cnt_in = jnp.sum(maskf)
    cnt0 = jnp.float32(DEG) - cnt_in
    term_ref[...] = row_sum + t_v_sum + cnt_in * bv_ref[...] - cnt0 * v0


def _tc_time_term(parts, v, times, thr, w0, b0, W, B, Wv, bv):
    return pl.pallas_call(
        _tc_body,
        grid=(1,),
        in_specs=[
            pl.BlockSpec((NW, HIDDEN), lambda i: (0, 0)),
            pl.BlockSpec((8, HIDDEN), lambda i: (0, 0)),
            pl.BlockSpec((DEG, 1), lambda i: (0, 0)),
            pl.BlockSpec((1, 1), lambda i: (0, 0)),
            pl.BlockSpec((1, 1), lambda i: (0, 0)),
            pl.BlockSpec((1, 1), lambda i: (0, 0)),
            pl.BlockSpec((1, 15), lambda i: (0, 0)),
            pl.BlockSpec((1, 15), lambda i: (0, 0)),
            pl.BlockSpec((16, HIDDEN), lambda i: (0, 0)),
            pl.BlockSpec((1, HIDDEN), lambda i: (0, 0)),
        ],
        out_specs=pl.BlockSpec((1, HIDDEN), lambda i: (0, 0)),
        out_shape=jax.ShapeDtypeStruct((1, HIDDEN), jnp.float32),
    )(parts, v, times, thr, w0, b0, W, B, Wv, bv)


def kernel(nid, k, q, v, t, neighbors, times,
           t2v_w0, t2v_b0, t2v_W, t2v_B, Wq, bq, Wk, bk, Wv, bv):
    del nid, k, q, Wq, bq, Wk, bk
    sel_masked = jnp.where(times.reshape(DEG) <= t[0],
                           neighbors.reshape(DEG).astype(jnp.int32), 0)
    parts = _sc_gather_sum(v, sel_masked)
    return _tc_time_term(
        parts, v, times.reshape(DEG, 1), t.reshape(1, 1), t2v_w0.reshape(1, 1),
        t2v_b0.reshape(1, 1), t2v_W.reshape(1, 15), t2v_B.reshape(1, 15),
        Wv, bv.reshape(1, HIDDEN))
